# baseline (speedup 1.0000x reference)
import functools

import jax
import jax.numpy as jnp
from jax import lax
from jax.experimental import pallas as pl
from jax.experimental.pallas import tpu as pltpu
from jax.experimental.pallas import tpu_sc as plsc

N = 10000
E = 320000
D = 128
NG = 64
EPS = 1e-5

DEGPAD = 10240
NT = 32
CHUNK = 128
NCH = 80
GCH = 16
NBUF = 2
NGRP = NCH // GCH
EPT = NCH * CHUNK
EPAD = NT * EPT
ROWS_MAIN = 632
DUMP_MAIN = (128, 128, 128, 128, 120)
DUMP_LAST = (128, 128, 128, 128, 8)

_mesh = plsc.VectorSubcoreMesh(core_axis_name="c", subcore_axis_name="s")


@functools.partial(
    pl.kernel,
    mesh=_mesh,
    out_type=(
        jax.ShapeDtypeStruct((2, DEGPAD), jnp.float32),
        jax.ShapeDtypeStruct((2, DEGPAD), jnp.float32),
    ),
    scratch_types=[
        pltpu.VMEM((NCH, CHUNK), jnp.int32),
        pltpu.VMEM((NCH, CHUNK), jnp.int32),
        pltpu.VMEM((NCH, CHUNK), jnp.float32),
        pltpu.VMEM((NCH, CHUNK), jnp.float32),
        pltpu.VMEM((640,), jnp.float32),
        pltpu.VMEM_SHARED((DEGPAD,), jnp.float32),
        pltpu.VMEM_SHARED((DEGPAD,), jnp.float32),
    ],
)
def _deg_kernel(rowp_hbm, colp_hbm, deg_hbm, s_hbm, ridx, cidx, ewb, swb, zsrc,
                dacc, sacc):
    cid = lax.axis_index("c")
    sid = lax.axis_index("s")
    wid = sid * 2 + cid

    def zb(i, c):
        zsrc[pl.ds(i * 16, 16)] = jnp.zeros((16,), jnp.float32)
        return c
    lax.fori_loop(0, 40, zb, 0)

    sl = pl.ds(sid * 640, 640)
    pltpu.sync_copy(zsrc, dacc.at[sl])
    pltpu.sync_copy(zsrc, sacc.at[sl])

    pltpu.sync_copy(rowp_hbm.at[wid], ridx)
    pltpu.sync_copy(colp_hbm.at[wid], cidx)

    def mk(j, c):
        for v in range(8):
            vs = pl.ds(v * 16, 16)
            loop = ridx[j, vs] == cidx[j, vs]
            ewb[j, vs] = jnp.where(loop, 0.0, 1.0)
            swb[j, vs] = jnp.where(loop, 1.0, 0.0)
        return c
    lax.fori_loop(0, NCH, mk, 0)

    plsc.subcore_barrier()

    def step(j, c):
        pltpu.sync_copy(ewb.at[j], dacc.at[ridx.at[j]], add=True)
        pltpu.sync_copy(swb.at[j], sacc.at[ridx.at[j]], add=True)
        return c
    lax.fori_loop(0, NCH, step, 0)

    plsc.subcore_barrier()

    pltpu.sync_copy(dacc.at[sl], deg_hbm.at[cid, sl])
    pltpu.sync_copy(sacc.at[sl], s_hbm.at[cid, sl])


@functools.partial(
    pl.kernel,
    mesh=_mesh,
    out_type=jax.ShapeDtypeStruct((2, N, D), jnp.float32),
    scratch_types=[
        pltpu.VMEM((2, GCH, CHUNK), jnp.int32),
        pltpu.VMEM((2, GCH, CHUNK), jnp.int32),
        pltpu.VMEM((NBUF, CHUNK, D), jnp.float32),
        pltpu.VMEM_SHARED((N, D), jnp.float32),
        pltpu.SemaphoreType.DMA,
        pltpu.SemaphoreType.DMA,
        pltpu.SemaphoreType.DMA,
    ],
)
def _scatter_kernel(u_hbm, rowp_hbm, colp_hbm, agg_hbm, ridx, cidx, gath, acc,
                    gsem, ssem, isem):
    cid = lax.axis_index("c")
    sid = lax.axis_index("s")
    wid = sid * 2 + cid

    def zrow(j, c):
        for v in range(8):
            gath[0, j, pl.ds(v * 16, 16)] = jnp.zeros((16,), jnp.float32)
        return c
    lax.fori_loop(0, CHUNK, zrow, 0)

    r0 = sid * ROWS_MAIN

    @pl.when(sid < 15)
    def _():
        for k, rk in enumerate(DUMP_MAIN):
            pltpu.sync_copy(gath.at[0, pl.ds(0, rk)],
                            acc.at[pl.ds(r0 + k * 128, rk)])

    @pl.when(sid == 15)
    def _():
        for k, rk in enumerate(DUMP_LAST):
            pltpu.sync_copy(gath.at[0, pl.ds(0, rk)],
                            acc.at[pl.ds(r0 + k * 128, rk)])

    plsc.subcore_barrier()

    def _g(p, j, b):
        return pltpu.async_copy(u_hbm.at[ridx.at[p, j]], gath.at[b], gsem)

    def _gw(p, j, b):
        pltpu.make_async_copy(u_hbm.at[ridx.at[p, j]], gath.at[b], gsem).wait()

    def _s(p, j, b):
        return pltpu.async_copy(gath.at[b], acc.at[cidx.at[p, j]], ssem,
                                add=True)

    def _sw(p, j, b):
        pltpu.make_async_copy(gath.at[b], acc.at[cidx.at[p, j]], ssem).wait()

    pltpu.sync_copy(rowp_hbm.at[wid, pl.ds(0, GCH)], ridx.at[0])
    pltpu.sync_copy(colp_hbm.at[wid, pl.ds(0, GCH)], cidx.at[0])

    for grp in range(NGRP):
        p = grp % 2
        q = 1 - p
        if grp > 0:
            nsl = pl.ds(grp * GCH, GCH)
            pltpu.make_async_copy(rowp_hbm.at[wid, nsl], ridx.at[p], isem).wait()
            pltpu.make_async_copy(colp_hbm.at[wid, nsl], cidx.at[p], isem).wait()
            for j in range(GCH - NBUF, GCH):
                _sw(q, j, j % NBUF)
        if grp + 1 < NGRP:
            nsl = pl.ds((grp + 1) * GCH, GCH)
            pltpu.async_copy(rowp_hbm.at[wid, nsl], ridx.at[q], isem)
            pltpu.async_copy(colp_hbm.at[wid, nsl], cidx.at[q], isem)
        _g(p, 0, 0)
        _g(p, 1, 1)
        for j in range(GCH):
            _gw(p, j, j % NBUF)
            _s(p, j, j % NBUF)
            if j + 2 < GCH:
                k = j + 2 - NBUF
                if k >= 0:
                    _sw(p, k, k % NBUF)
                _g(p, j + 2, (j + 2) % NBUF)

    for j in range(GCH - NBUF, GCH):
        _sw((NGRP - 1) % 2, j, j % NBUF)

    plsc.subcore_barrier()

    @pl.when(sid < 15)
    def _():
        sl = pl.ds(r0, ROWS_MAIN)
        pltpu.sync_copy(acc.at[sl], agg_hbm.at[cid, sl])

    @pl.when(sid == 15)
    def _():
        sl = pl.ds(r0, 520)
        pltpu.sync_copy(acc.at[sl], agg_hbm.at[cid, sl])


def _feat_body(x_ref, g_ref, b_ref, w_ref, h_ref):
    x = x_ref[...]
    m = jnp.mean(x, axis=0, keepdims=True)
    d = x - m
    v = jnp.mean(d * d, axis=0, keepdims=True)
    hn = d * lax.rsqrt(v + EPS) * g_ref[...] + b_ref[...]
    h_ref[...] = jnp.maximum(
        jnp.dot(hn, w_ref[...], preferred_element_type=jnp.float32), 0.0)


_feat_kernel = pl.pallas_call(
    _feat_body,
    out_shape=jax.ShapeDtypeStruct((N, D), jnp.float32),
)


def _dinv_body(degp_ref, sp_ref, dinv_ref, s_ref):
    dp = degp_ref[...]
    dinv_ref[...] = lax.rsqrt(dp[0:1, :] + dp[1:2, :] + 1.0)
    sp = sp_ref[...]
    s_ref[...] = sp[0:1, :] + sp[1:2, :]


_dinv_kernel = pl.pallas_call(
    _dinv_body,
    out_shape=(
        jax.ShapeDtypeStruct((1, DEGPAD), jnp.float32),
        jax.ShapeDtypeStruct((1, DEGPAD), jnp.float32),
    ),
)


def _dense_body(h_ref, g_ref, b_ref, w_ref, dinv_ref, gout_ref, uout_ref):
    h = h_ref[...]
    m = jnp.mean(h, axis=0, keepdims=True)
    d = h - m
    v = jnp.mean(d * d, axis=0, keepdims=True)
    hn = d * lax.rsqrt(v + EPS) * g_ref[...] + b_ref[...]
    g = jnp.dot(hn, w_ref[...], preferred_element_type=jnp.float32)
    gout_ref[...] = g
    uout_ref[...] = g * dinv_ref[...]


_dense_kernel = pl.pallas_call(
    _dense_body,
    out_shape=(
        jax.ShapeDtypeStruct((N, D), jnp.float32),
        jax.ShapeDtypeStruct((N, D), jnp.float32),
    ),
)


def _fused_body(hp_ref, gp_ref, a0_ref, a1_ref, dinv_ref, s_ref,
                bp_ref, g_ref, b_ref, w_ref, hout_ref, gout_ref, uout_ref):
    dinv = dinv_ref[...]
    t = ((a0_ref[...] + a1_ref[...]) * dinv
         + gp_ref[...] * ((1.0 - s_ref[...]) * dinv * dinv) + bp_ref[...])
    h = hp_ref[...] + jnp.maximum(t, 0.0)
    hout_ref[...] = h
    m = jnp.mean(h, axis=0, keepdims=True)
    d = h - m
    v = jnp.mean(d * d, axis=0, keepdims=True)
    hn = d * lax.rsqrt(v + EPS) * g_ref[...] + b_ref[...]
    g = jnp.dot(hn, w_ref[...], preferred_element_type=jnp.float32)
    gout_ref[...] = g
    uout_ref[...] = g * dinv


_fused_kernel = pl.pallas_call(
    _fused_body,
    out_shape=(
        jax.ShapeDtypeStruct((N, D), jnp.float32),
        jax.ShapeDtypeStruct((N, D), jnp.float32),
        jax.ShapeDtypeStruct((N, D), jnp.float32),
    ),
)


def _head_body(hp_ref, gp_ref, a0_ref, a1_ref, dinv_ref, s_ref, bp_ref,
               batch_ref, g4_ref, b4_ref, fcw_ref, fcb_ref, g5_ref, b5_ref,
               out_ref):
    dinv = dinv_ref[...]
    t = ((a0_ref[...] + a1_ref[...]) * dinv
         + gp_ref[...] * ((1.0 - s_ref[...]) * dinv * dinv) + bp_ref[...])
    h = hp_ref[...] + jnp.maximum(t, 0.0)
    gid = lax.broadcasted_iota(jnp.int32, (NG, N), 0)
    oh = (gid == batch_ref[...]).astype(jnp.float32)
    xg = jnp.dot(oh, h, preferred_element_type=jnp.float32)
    m = jnp.mean(xg, axis=0, keepdims=True)
    d = xg - m
    v = jnp.mean(d * d, axis=0, keepdims=True)
    xn = d * lax.rsqrt(v + EPS) * g4_ref[...] + b4_ref[...]
    x_ = jnp.maximum(
        jnp.dot(xn, fcw_ref[...], preferred_element_type=jnp.float32)
        + fcb_ref[...], 0.0)
    m2 = jnp.mean(x_, axis=0, keepdims=True)
    d2 = x_ - m2
    v2 = jnp.mean(d2 * d2, axis=0, keepdims=True)
    out_ref[...] = d2 * lax.rsqrt(v2 + EPS) * g5_ref[...] + b5_ref[...]


_head_kernel = pl.pallas_call(
    _head_body,
    out_shape=jax.ShapeDtypeStruct((NG, D), jnp.float32),
)


def kernel(x, edge_index, batch, W_feat, conv_W, conv_b, fc_W, fc_b, bn_gamma, bn_beta):
    row = edge_index[0]
    col = edge_index[1]
    pad = (jnp.arange(EPAD - E, dtype=jnp.int32) % N)
    rowp = jnp.concatenate([row, pad]).reshape(NT, NCH, CHUNK)
    colp = jnp.concatenate([col, pad]).reshape(NT, NCH, CHUNK)

    degp, sp = _deg_kernel(rowp, colp)
    h = _feat_kernel(x, bn_gamma[0:1], bn_beta[0:1], W_feat)
    dinv_row, s_row = _dinv_kernel(degp, sp)
    dinv = dinv_row.reshape(DEGPAD, 1)[:N]
    s = s_row.reshape(DEGPAD, 1)[:N]

    g, u = _dense_kernel(h, bn_gamma[1:2], bn_beta[1:2], conv_W[0], dinv)
    aggp = _scatter_kernel(u, rowp, colp)

    for i in (1, 2):
        h, g, u = _fused_kernel(h, g, aggp[0], aggp[1], dinv, s,
                                conv_b[i - 1:i], bn_gamma[1 + i:2 + i],
                                bn_beta[1 + i:2 + i], conv_W[i])
        aggp = _scatter_kernel(u, rowp, colp)

    return _head_kernel(h, g, aggp[0], aggp[1], dinv, s, conv_b[2:3],
                        batch.reshape(1, N), bn_gamma[4:5], bn_beta[4:5],
                        fc_W, fc_b.reshape(1, D), bn_gamma[5:6], bn_beta[5:6])


# speedup vs baseline: 1.0219x; 1.0219x over previous
import functools

import jax
import jax.numpy as jnp
from jax import lax
from jax.experimental import pallas as pl
from jax.experimental.pallas import tpu as pltpu
from jax.experimental.pallas import tpu_sc as plsc

N = 10000
E = 320000
D = 128
NG = 64
EPS = 1e-5

DEGPAD = 10240
NT = 32
CHUNK = 128
NCH = 80
GCH = 16
NBUF = 2
NGRP = NCH // GCH
EPT = NCH * CHUNK
EPAD = NT * EPT
ROWS_MAIN = 632
DUMP_MAIN = (128, 128, 128, 128, 120)
DUMP_LAST = (128, 128, 128, 128, 8)

_mesh = plsc.VectorSubcoreMesh(core_axis_name="c", subcore_axis_name="s")


@functools.partial(
    pl.kernel,
    mesh=_mesh,
    out_type=(
        jax.ShapeDtypeStruct((2, DEGPAD), jnp.float32),
        jax.ShapeDtypeStruct((2, DEGPAD), jnp.float32),
    ),
    scratch_types=[
        pltpu.VMEM((NCH, CHUNK), jnp.int32),
        pltpu.VMEM((NCH, CHUNK), jnp.int32),
        pltpu.VMEM((NCH, CHUNK), jnp.float32),
        pltpu.VMEM((NCH, CHUNK), jnp.float32),
        pltpu.VMEM((640,), jnp.float32),
        pltpu.VMEM_SHARED((DEGPAD,), jnp.float32),
        pltpu.VMEM_SHARED((DEGPAD,), jnp.float32),
        pltpu.SemaphoreType.DMA,
    ],
)
def _deg_kernel(rowp_hbm, colp_hbm, deg_hbm, s_hbm, ridx, cidx, ewb, swb, zsrc,
                dacc, sacc, dsem):
    cid = lax.axis_index("c")
    sid = lax.axis_index("s")
    wid = sid * 2 + cid

    def zb(i, c):
        zsrc[pl.ds(i * 16, 16)] = jnp.zeros((16,), jnp.float32)
        return c
    lax.fori_loop(0, 40, zb, 0)

    sl = pl.ds(sid * 640, 640)
    pltpu.sync_copy(zsrc, dacc.at[sl])
    pltpu.sync_copy(zsrc, sacc.at[sl])

    pltpu.sync_copy(rowp_hbm.at[wid], ridx)
    pltpu.sync_copy(colp_hbm.at[wid], cidx)

    def mk(j, c):
        for v in range(8):
            vs = pl.ds(v * 16, 16)
            loop = ridx[j, vs] == cidx[j, vs]
            ewb[j, vs] = jnp.where(loop, 0.0, 1.0)
            swb[j, vs] = jnp.where(loop, 1.0, 0.0)
        return c
    lax.fori_loop(0, NCH, mk, 0)

    plsc.subcore_barrier()

    def step(k, c):
        def fire(j, c2):
            pltpu.async_copy(ewb.at[j], dacc.at[ridx.at[j]], dsem, add=True)
            pltpu.async_copy(swb.at[j], sacc.at[ridx.at[j]], dsem, add=True)
            return c2
        lax.fori_loop(k * 8, (k + 1) * 8, fire, 0)

        def drain(j, c2):
            pltpu.make_async_copy(ewb.at[j], dacc.at[ridx.at[j]], dsem).wait()
            pltpu.make_async_copy(swb.at[j], sacc.at[ridx.at[j]], dsem).wait()
            return c2
        lax.fori_loop(k * 8, (k + 1) * 8, drain, 0)
        return c
    lax.fori_loop(0, NCH // 8, step, 0)

    plsc.subcore_barrier()

    pltpu.sync_copy(dacc.at[sl], deg_hbm.at[cid, sl])
    pltpu.sync_copy(sacc.at[sl], s_hbm.at[cid, sl])


@functools.partial(
    pl.kernel,
    mesh=_mesh,
    out_type=jax.ShapeDtypeStruct((2, N, D), jnp.float32),
    scratch_types=[
        pltpu.VMEM((2, GCH, CHUNK), jnp.int32),
        pltpu.VMEM((2, GCH, CHUNK), jnp.int32),
        pltpu.VMEM((NBUF, CHUNK, D), jnp.float32),
        pltpu.VMEM_SHARED((N, D), jnp.float32),
        pltpu.SemaphoreType.DMA,
        pltpu.SemaphoreType.DMA,
        pltpu.SemaphoreType.DMA,
    ],
)
def _scatter_kernel(u_hbm, rowp_hbm, colp_hbm, agg_hbm, ridx, cidx, gath, acc,
                    gsem, ssem, isem):
    cid = lax.axis_index("c")
    sid = lax.axis_index("s")
    wid = sid * 2 + cid

    def zrow(j, c):
        for v in range(8):
            gath[0, j, pl.ds(v * 16, 16)] = jnp.zeros((16,), jnp.float32)
        return c
    lax.fori_loop(0, CHUNK, zrow, 0)

    r0 = sid * ROWS_MAIN

    @pl.when(sid < 15)
    def _():
        for k, rk in enumerate(DUMP_MAIN):
            pltpu.sync_copy(gath.at[0, pl.ds(0, rk)],
                            acc.at[pl.ds(r0 + k * 128, rk)])

    @pl.when(sid == 15)
    def _():
        for k, rk in enumerate(DUMP_LAST):
            pltpu.sync_copy(gath.at[0, pl.ds(0, rk)],
                            acc.at[pl.ds(r0 + k * 128, rk)])

    plsc.subcore_barrier()

    def _g(p, j, b):
        return pltpu.async_copy(u_hbm.at[ridx.at[p, j]], gath.at[b], gsem)

    def _gw(p, j, b):
        pltpu.make_async_copy(u_hbm.at[ridx.at[p, j]], gath.at[b], gsem).wait()

    def _s(p, j, b):
        return pltpu.async_copy(gath.at[b], acc.at[cidx.at[p, j]], ssem,
                                add=True)

    def _sw(p, j, b):
        pltpu.make_async_copy(gath.at[b], acc.at[cidx.at[p, j]], ssem).wait()

    pltpu.sync_copy(rowp_hbm.at[wid, pl.ds(0, GCH)], ridx.at[0])
    pltpu.sync_copy(colp_hbm.at[wid, pl.ds(0, GCH)], cidx.at[0])

    for grp in range(NGRP):
        p = grp % 2
        q = 1 - p
        if grp > 0:
            nsl = pl.ds(grp * GCH, GCH)
            pltpu.make_async_copy(rowp_hbm.at[wid, nsl], ridx.at[p], isem).wait()
            pltpu.make_async_copy(colp_hbm.at[wid, nsl], cidx.at[p], isem).wait()
            for j in range(GCH - NBUF, GCH):
                _sw(q, j, j % NBUF)
        if grp + 1 < NGRP:
            nsl = pl.ds((grp + 1) * GCH, GCH)
            pltpu.async_copy(rowp_hbm.at[wid, nsl], ridx.at[q], isem)
            pltpu.async_copy(colp_hbm.at[wid, nsl], cidx.at[q], isem)
        _g(p, 0, 0)
        _g(p, 1, 1)
        for j in range(GCH):
            _gw(p, j, j % NBUF)
            _s(p, j, j % NBUF)
            if j + 2 < GCH:
                k = j + 2 - NBUF
                if k >= 0:
                    _sw(p, k, k % NBUF)
                _g(p, j + 2, (j + 2) % NBUF)

    for j in range(GCH - NBUF, GCH):
        _sw((NGRP - 1) % 2, j, j % NBUF)

    plsc.subcore_barrier()

    @pl.when(sid < 15)
    def _():
        sl = pl.ds(r0, ROWS_MAIN)
        pltpu.sync_copy(acc.at[sl], agg_hbm.at[cid, sl])

    @pl.when(sid == 15)
    def _():
        sl = pl.ds(r0, 520)
        pltpu.sync_copy(acc.at[sl], agg_hbm.at[cid, sl])


def _feat_body(x_ref, g_ref, b_ref, w_ref, h_ref):
    x = x_ref[...]
    m = jnp.mean(x, axis=0, keepdims=True)
    d = x - m
    v = jnp.mean(d * d, axis=0, keepdims=True)
    hn = d * lax.rsqrt(v + EPS) * g_ref[...] + b_ref[...]
    h_ref[...] = jnp.maximum(
        jnp.dot(hn, w_ref[...], preferred_element_type=jnp.float32), 0.0)


_feat_kernel = pl.pallas_call(
    _feat_body,
    out_shape=jax.ShapeDtypeStruct((N, D), jnp.float32),
)


def _dinv_body(degp_ref, sp_ref, dinv_ref, s_ref):
    dp = degp_ref[...]
    dinv_ref[...] = lax.rsqrt(dp[0:1, :] + dp[1:2, :] + 1.0)
    sp = sp_ref[...]
    s_ref[...] = sp[0:1, :] + sp[1:2, :]


_dinv_kernel = pl.pallas_call(
    _dinv_body,
    out_shape=(
        jax.ShapeDtypeStruct((1, DEGPAD), jnp.float32),
        jax.ShapeDtypeStruct((1, DEGPAD), jnp.float32),
    ),
)


def _dense_body(h_ref, g_ref, b_ref, w_ref, dinv_ref, gout_ref, uout_ref):
    h = h_ref[...]
    m = jnp.mean(h, axis=0, keepdims=True)
    d = h - m
    v = jnp.mean(d * d, axis=0, keepdims=True)
    hn = d * lax.rsqrt(v + EPS) * g_ref[...] + b_ref[...]
    g = jnp.dot(hn, w_ref[...], preferred_element_type=jnp.float32)
    gout_ref[...] = g
    uout_ref[...] = g * dinv_ref[...]


_dense_kernel = pl.pallas_call(
    _dense_body,
    out_shape=(
        jax.ShapeDtypeStruct((N, D), jnp.float32),
        jax.ShapeDtypeStruct((N, D), jnp.float32),
    ),
)


def _fused_body(hp_ref, gp_ref, a0_ref, a1_ref, dinv_ref, s_ref,
                bp_ref, g_ref, b_ref, w_ref, hout_ref, gout_ref, uout_ref):
    dinv = dinv_ref[...]
    t = ((a0_ref[...] + a1_ref[...]) * dinv
         + gp_ref[...] * ((1.0 - s_ref[...]) * dinv * dinv) + bp_ref[...])
    h = hp_ref[...] + jnp.maximum(t, 0.0)
    hout_ref[...] = h
    m = jnp.mean(h, axis=0, keepdims=True)
    d = h - m
    v = jnp.mean(d * d, axis=0, keepdims=True)
    hn = d * lax.rsqrt(v + EPS) * g_ref[...] + b_ref[...]
    g = jnp.dot(hn, w_ref[...], preferred_element_type=jnp.float32)
    gout_ref[...] = g
    uout_ref[...] = g * dinv


_fused_kernel = pl.pallas_call(
    _fused_body,
    out_shape=(
        jax.ShapeDtypeStruct((N, D), jnp.float32),
        jax.ShapeDtypeStruct((N, D), jnp.float32),
        jax.ShapeDtypeStruct((N, D), jnp.float32),
    ),
)


def _head_body(hp_ref, gp_ref, a0_ref, a1_ref, dinv_ref, s_ref, bp_ref,
               batch_ref, g4_ref, b4_ref, fcw_ref, fcb_ref, g5_ref, b5_ref,
               out_ref):
    dinv = dinv_ref[...]
    t = ((a0_ref[...] + a1_ref[...]) * dinv
         + gp_ref[...] * ((1.0 - s_ref[...]) * dinv * dinv) + bp_ref[...])
    h = hp_ref[...] + jnp.maximum(t, 0.0)
    gid = lax.broadcasted_iota(jnp.int32, (NG, N), 0)
    oh = (gid == batch_ref[...]).astype(jnp.float32)
    xg = jnp.dot(oh, h, preferred_element_type=jnp.float32)
    m = jnp.mean(xg, axis=0, keepdims=True)
    d = xg - m
    v = jnp.mean(d * d, axis=0, keepdims=True)
    xn = d * lax.rsqrt(v + EPS) * g4_ref[...] + b4_ref[...]
    x_ = jnp.maximum(
        jnp.dot(xn, fcw_ref[...], preferred_element_type=jnp.float32)
        + fcb_ref[...], 0.0)
    m2 = jnp.mean(x_, axis=0, keepdims=True)
    d2 = x_ - m2
    v2 = jnp.mean(d2 * d2, axis=0, keepdims=True)
    out_ref[...] = d2 * lax.rsqrt(v2 + EPS) * g5_ref[...] + b5_ref[...]


_head_kernel = pl.pallas_call(
    _head_body,
    out_shape=jax.ShapeDtypeStruct((NG, D), jnp.float32),
)


def kernel(x, edge_index, batch, W_feat, conv_W, conv_b, fc_W, fc_b, bn_gamma, bn_beta):
    row = edge_index[0]
    col = edge_index[1]
    pad = (jnp.arange(EPAD - E, dtype=jnp.int32) % N)
    rowp = jnp.concatenate([row, pad]).reshape(NT, NCH, CHUNK)
    colp = jnp.concatenate([col, pad]).reshape(NT, NCH, CHUNK)

    degp, sp = _deg_kernel(rowp, colp)
    h = _feat_kernel(x, bn_gamma[0:1], bn_beta[0:1], W_feat)
    dinv_row, s_row = _dinv_kernel(degp, sp)
    dinv = dinv_row.reshape(DEGPAD, 1)[:N]
    s = s_row.reshape(DEGPAD, 1)[:N]

    g, u = _dense_kernel(h, bn_gamma[1:2], bn_beta[1:2], conv_W[0], dinv)
    aggp = _scatter_kernel(u, rowp, colp)

    for i in (1, 2):
        h, g, u = _fused_kernel(h, g, aggp[0], aggp[1], dinv, s,
                                conv_b[i - 1:i], bn_gamma[1 + i:2 + i],
                                bn_beta[1 + i:2 + i], conv_W[i])
        aggp = _scatter_kernel(u, rowp, colp)

    return _head_kernel(h, g, aggp[0], aggp[1], dinv, s, conv_b[2:3],
                        batch.reshape(1, N), bn_gamma[4:5], bn_beta[4:5],
                        fc_W, fc_b.reshape(1, D), bn_gamma[5:6], bn_beta[5:6])
